# baseline (speedup 1.0000x reference)
import functools

import jax
import jax.numpy as jnp
from jax import lax
from jax.experimental import pallas as pl
from jax.experimental.pallas import tpu as pltpu
from jax.experimental.pallas import tpu_sc as plsc


def _sc_gather(flat_src, table, *, num_workers, chunk):
    B = flat_src.shape[0]
    D = table.shape[1]
    b_per_w = B // num_workers
    nchunks = b_per_w // chunk

    assert nchunks % 2 == 0
    mesh = plsc.VectorSubcoreMesh(core_axis_name="c", subcore_axis_name="s")

    @functools.partial(
        pl.kernel,
        mesh=mesh,
        out_type=jax.ShapeDtypeStruct((B, D), jnp.float32),
        scratch_types=[
            pltpu.VMEM((chunk,), jnp.int32),
            pltpu.VMEM((chunk,), jnp.int32),
            pltpu.VMEM((chunk, D), jnp.float32),
            pltpu.VMEM((chunk, D), jnp.float32),
            pltpu.SemaphoreType.DMA,
            pltpu.SemaphoreType.DMA,
            pltpu.SemaphoreType.DMA,
            pltpu.SemaphoreType.DMA,
        ],
        compiler_params=pltpu.CompilerParams(use_tc_tiling_on_sc=False),
    )
    def emb_kernel(
        src_hbm, table_hbm, out_hbm,
        idx0, idx1, rows0, rows1, gsem0, gsem1, ssem0, ssem1,
    ):
        wid = lax.axis_index("s") * 2 + lax.axis_index("c")
        wbase = wid * b_per_w
        bufs = ((idx0, rows0, gsem0, ssem0), (idx1, rows1, gsem1, ssem1))

        for b, (idx_v, rows_v, gsem, _) in enumerate(bufs):
            pltpu.sync_copy(src_hbm.at[pl.ds(wbase + b * chunk, chunk)], idx_v)
            pltpu.async_copy(table_hbm.at[idx_v], rows_v, gsem)

        def pair_body(h, carry):
            for b, (idx_v, rows_v, gsem, ssem) in enumerate(bufs):
                g = h * 2 + b
                base = wbase + g * chunk
                pltpu.make_async_copy(table_hbm.at[idx_v], rows_v, gsem).wait()
                pltpu.async_copy(rows_v, out_hbm.at[pl.ds(base, chunk)], ssem)

                @pl.when(g + 2 < nchunks)
                def _():
                    nbase = base + 2 * chunk
                    pltpu.make_async_copy(
                        rows_v, out_hbm.at[pl.ds(base, chunk)], ssem
                    ).wait()
                    pltpu.sync_copy(src_hbm.at[pl.ds(nbase, chunk)], idx_v)
                    pltpu.async_copy(table_hbm.at[idx_v], rows_v, gsem)

            return carry

        lax.fori_loop(0, nchunks // 2, pair_body, 0)
        for b, (idx_v, rows_v, _, ssem) in enumerate(bufs):
            base = wbase + (nchunks - 2 + b) * chunk
            pltpu.make_async_copy(
                rows_v, out_hbm.at[pl.ds(base, chunk)], ssem
            ).wait()

    return emb_kernel(flat_src, table)


def kernel(src, table):
    V, D = table.shape
    B, T = src.shape
    flat = src.T.reshape(-1).astype(jnp.int32)
    out = _sc_gather(flat, table, num_workers=32, chunk=1280)
    om = out.reshape(T, B, D)
    mid = jax.lax.optimization_barrier(om.transpose(0, 2, 1))
    return mid.transpose(2, 0, 1)


# speedup vs baseline: 1.0050x; 1.0050x over previous
import functools

import jax
import jax.numpy as jnp
from jax import lax
from jax.experimental import pallas as pl
from jax.experimental.pallas import tpu as pltpu
from jax.experimental.pallas import tpu_sc as plsc


def _sc_gather(flat_src, table, *, num_workers, chunk):
    B = flat_src.shape[0]
    D = table.shape[1]
    b_per_w = B // num_workers
    nchunks = b_per_w // chunk

    assert nchunks % 2 == 0
    mesh = plsc.VectorSubcoreMesh(core_axis_name="c", subcore_axis_name="s")

    @functools.partial(
        pl.kernel,
        mesh=mesh,
        out_type=jax.ShapeDtypeStruct((B, D), jnp.float32),
        scratch_types=[
            pltpu.VMEM((chunk,), jnp.int32),
            pltpu.VMEM((chunk,), jnp.int32),
            pltpu.VMEM((chunk, D), jnp.float32),
            pltpu.VMEM((chunk, D), jnp.float32),
            pltpu.SemaphoreType.DMA,
            pltpu.SemaphoreType.DMA,
            pltpu.SemaphoreType.DMA,
            pltpu.SemaphoreType.DMA,
        ],
        compiler_params=pltpu.CompilerParams(use_tc_tiling_on_sc=False),
    )
    def emb_kernel(
        src_hbm, table_hbm, out_hbm,
        idx0, idx1, rows0, rows1, gsem0, gsem1, ssem0, ssem1,
    ):
        wid = lax.axis_index("s") * 2 + lax.axis_index("c")
        wbase = wid * b_per_w
        bufs = ((idx0, rows0, gsem0, ssem0), (idx1, rows1, gsem1, ssem1))

        for b, (idx_v, rows_v, gsem, _) in enumerate(bufs):
            pltpu.sync_copy(src_hbm.at[pl.ds(wbase + b * chunk, chunk)], idx_v)
            pltpu.async_copy(table_hbm.at[idx_v], rows_v, gsem)

        def pair_body(h, carry):
            for b, (idx_v, rows_v, gsem, ssem) in enumerate(bufs):
                g = h * 2 + b
                base = wbase + g * chunk
                pltpu.make_async_copy(table_hbm.at[idx_v], rows_v, gsem).wait()
                pltpu.async_copy(rows_v, out_hbm.at[pl.ds(base, chunk)], ssem)

                @pl.when(g + 2 < nchunks)
                def _():
                    nbase = base + 2 * chunk
                    pltpu.make_async_copy(
                        rows_v, out_hbm.at[pl.ds(base, chunk)], ssem
                    ).wait()
                    pltpu.sync_copy(src_hbm.at[pl.ds(nbase, chunk)], idx_v)
                    pltpu.async_copy(table_hbm.at[idx_v], rows_v, gsem)

            return carry

        lax.fori_loop(0, nchunks // 2, pair_body, 0)
        for b, (idx_v, rows_v, _, ssem) in enumerate(bufs):
            base = wbase + (nchunks - 2 + b) * chunk
            pltpu.make_async_copy(
                rows_v, out_hbm.at[pl.ds(base, chunk)], ssem
            ).wait()

    return emb_kernel(flat_src, table)


def kernel(src, table):
    V, D = table.shape
    B, T = src.shape
    flat = src.T.reshape(-1).astype(jnp.int32)
    out = _sc_gather(flat, table, num_workers=32, chunk=1600)
    om = out.reshape(T, B, D)
    mid = jax.lax.optimization_barrier(om.transpose(0, 2, 1))
    return mid.transpose(2, 0, 1)
